# baseline (speedup 1.0000x reference)
import jax
import jax.numpy as jnp
from jax import lax
from jax.experimental import pallas as pl
from jax.experimental.pallas import tpu as pltpu
from jax.experimental.pallas import tpu_sc as plsc

B = 16384
K = 32
NC = 2
NS = 16
NW = NC * NS
BPW = B // NW
CH = 128
NCH = BPW // CH
ROWS = 250000


def _mf_body(du_hbm, di_hbm, ur_hbm, ir_hbm, ub_hbm, ib_hbm, out_hbm,
             idx_u, idx_i, q_u, q_i, rl_u, rl_i, ubuf, ibuf, ubv, ibv,
             outv, sem_u0, sem_u1, sem_i0, sem_i1, sem_b):
    wid = lax.axis_index("s") * NC + lax.axis_index("c")

    pltpu.sync_copy(du_hbm.at[wid], idx_u)
    pltpu.sync_copy(di_hbm.at[wid], idx_i)

    for c in range(NCH):
        for j in range(CH // 16):
            s = pl.ds(j * 16, 16)
            vu = idx_u[c, s]
            vi = idx_i[c, s]
            q_u[c, s] = lax.shift_right_logical(vu, 2)
            q_i[c, s] = lax.shift_right_logical(vi, 2)
            rl_u[c, s] = lax.shift_left(jnp.bitwise_and(vu, 3), 5)
            rl_i[c, s] = lax.shift_left(jnp.bitwise_and(vi, 3), 5)

    bias_copies = []
    for c in range(NCH):
        bias_copies.append(
            pltpu.async_copy(ub_hbm.at[idx_u.at[c]], ubv.at[c], sem_b))
        bias_copies.append(
            pltpu.async_copy(ib_hbm.at[idx_i.at[c]], ibv.at[c], sem_b))

    sems = [(sem_u0, sem_i0), (sem_u1, sem_i1)]

    def fire(c):
        pb = c & 1
        su, si = sems[pb]
        cu = pltpu.async_copy(ur_hbm.at[q_u.at[c]], ubuf.at[pb], su)
        ci = pltpu.async_copy(ir_hbm.at[q_i.at[c]], ibuf.at[pb], si)
        return (cu, ci)

    lane = lax.iota(jnp.int32, 16)
    inflight = {0: fire(0)}
    for c in range(NCH):
        pb = c & 1
        cu, ci = inflight.pop(c)
        cu.wait()
        ci.wait()
        if c + 1 < NCH:
            inflight[c + 1] = fire(c + 1)
        for g in range(CH // 16):
            s = pl.ds(g * 16, 16)
            rows = g * 16 + lane
            cu_cols = rl_u[c, s]
            ci_cols = rl_i[c, s]
            acc = ubv[c, s] + ibv[c, s]
            for t in range(K):
                perm = jnp.bitwise_and(lane + t, K - 1)
                uu = plsc.load_gather(ubuf.at[pb], [rows, cu_cols + perm])
                ii = plsc.load_gather(ibuf.at[pb], [rows, ci_cols + perm])
                acc = acc + uu * ii
            outv[pl.ds(c * CH + g * 16, 16)] = acc

    for cp in bias_copies:
        cp.wait()

    pltpu.sync_copy(outv, out_hbm.at[pl.ds(wid * BPW, BPW)])


@jax.jit
def _mf(du, di, ur, ir, ub, ib):
    mesh = plsc.VectorSubcoreMesh(core_axis_name="c", subcore_axis_name="s")
    return pl.kernel(
        _mf_body,
        out_type=jax.ShapeDtypeStruct((B,), jnp.float32),
        mesh=mesh,
        compiler_params=pltpu.CompilerParams(
            needs_layout_passes=False, use_tc_tiling_on_sc=True),
        scratch_types=[
            pltpu.VMEM((NCH, CH), jnp.int32),
            pltpu.VMEM((NCH, CH), jnp.int32),
            pltpu.VMEM((NCH, CH), jnp.int32),
            pltpu.VMEM((NCH, CH), jnp.int32),
            pltpu.VMEM((NCH, CH), jnp.int32),
            pltpu.VMEM((NCH, CH), jnp.int32),
            pltpu.VMEM((2, CH, 4 * K), jnp.float32),
            pltpu.VMEM((2, CH, 4 * K), jnp.float32),
            pltpu.VMEM((NCH, CH), jnp.float32),
            pltpu.VMEM((NCH, CH), jnp.float32),
            pltpu.VMEM((BPW,), jnp.float32),
            pltpu.SemaphoreType.DMA,
            pltpu.SemaphoreType.DMA,
            pltpu.SemaphoreType.DMA,
            pltpu.SemaphoreType.DMA,
            pltpu.SemaphoreType.DMA,
        ],
    )(du, di, ur, ir, ub, ib)


def kernel(data_u, data_i, u_emb, i_emb, user_b, item_b):
    du = data_u.astype(jnp.int32).reshape(NW, NCH, CH)
    di = data_i.astype(jnp.int32).reshape(NW, NCH, CH)
    ur = u_emb.reshape(ROWS, 4 * K)
    ir = i_emb.reshape(ROWS, 4 * K)
    ub = user_b.reshape(-1)
    ib = item_b.reshape(-1)
    return _mf(du, di, ur, ir, ub, ib)
